# baseline (speedup 1.0000x reference)
import jax
import jax.numpy as jnp
from jax.experimental import pallas as pl
from jax.experimental.pallas import tpu as pltpu

_D = 128
_N = 100
_B = 1024
_BB = 8
_TEMP = 1.0


def _body(env_idx_ref, w_ref, deltas_ref, a_ref, wb_ref, ta_ref):
    i = pl.program_id(0)

    @pl.when(i == 0)
    def _tables():
        w = w_ref[...]
        row = jax.lax.broadcasted_iota(jnp.int32, (_D, _D), 0)
        col = jax.lax.broadcasted_iota(jnp.int32, (_D, _D), 1)
        mask = jnp.where(row == col, 0.0, 1.0)
        ta_ref[0:_N] = jax.nn.sigmoid((w[None] + deltas_ref[...]) * (1.0 / _TEMP)) * mask[None]
        ta_ref[_N] = jax.nn.sigmoid(w * (1.0 / _TEMP)) * mask

    @pl.when(i > 0)
    def _emit():
        w = w_ref[...]
        base = (i - 1) * _BB
        for j in range(_BB):
            e = env_idx_ref[base + j]
            idx = jnp.clip(e, 0, _N - 1)
            valid = e < _N
            wb_ref[j] = w + jnp.where(valid, 1.0, 0.0) * deltas_ref[idx]
            a_ref[j] = ta_ref[jnp.where(valid, idx, _N)]


@jax.jit
def _run(env_idx, W_adj, env_deltas):
    grid = (1 + _B // _BB,)
    out_shape = (
        jax.ShapeDtypeStruct((_B, _D, _D), jnp.float32),
        jax.ShapeDtypeStruct((_B, _D, _D), jnp.float32),
    )
    out_map = lambda i: (jnp.maximum(i - 1, 0), 0, 0)
    return pl.pallas_call(
        _body,
        grid=grid,
        in_specs=[
            pl.BlockSpec(memory_space=pltpu.SMEM),
            pl.BlockSpec((_D, _D), lambda i: (0, 0)),
            pl.BlockSpec((_N, _D, _D), lambda i: (0, 0, 0)),
        ],
        out_specs=[
            pl.BlockSpec((_BB, _D, _D), out_map),
            pl.BlockSpec((_BB, _D, _D), out_map),
        ],
        out_shape=out_shape,
        scratch_shapes=[pltpu.VMEM((_N + 1, _D, _D), jnp.float32)],
    )(env_idx, W_adj, env_deltas)


def kernel(env_idx, W_adj, env_deltas):
    return _run(env_idx, W_adj, env_deltas)


# speedup vs baseline: 1.3998x; 1.3998x over previous
import jax
import jax.numpy as jnp
from jax.experimental import pallas as pl
from jax.experimental.pallas import tpu as pltpu

_D = 128
_N = 100
_B = 1024
_BB = 16
_TEMP = 1.0


def _body(env_idx_ref, w_ref, deltas_ref, a_ref, wb_ref, ta_ref):
    i = pl.program_id(0)

    @pl.when(i == 0)
    def _tables():
        w = w_ref[...]
        row = jax.lax.broadcasted_iota(jnp.int32, (_D, _D), 0)
        col = jax.lax.broadcasted_iota(jnp.int32, (_D, _D), 1)
        mask = jnp.where(row == col, 0.0, 1.0)
        ta_ref[0:_N] = jax.nn.sigmoid((w[None] + deltas_ref[...]) * (1.0 / _TEMP)) * mask[None]
        ta_ref[_N] = jax.nn.sigmoid(w * (1.0 / _TEMP)) * mask

    @pl.when(i > 0)
    def _emit():
        w = w_ref[...]
        base = (i - 1) * _BB
        for j in range(_BB):
            e = env_idx_ref[base + j]
            idx = jnp.clip(e, 0, _N - 1)
            valid = e < _N
            wb_ref[j] = w + jnp.where(valid, 1.0, 0.0) * deltas_ref[idx]
            a_ref[j] = ta_ref[jnp.where(valid, idx, _N)]


@jax.jit
def _run(env_idx, W_adj, env_deltas):
    grid = (1 + _B // _BB,)
    out_shape = (
        jax.ShapeDtypeStruct((_B, _D, _D), jnp.float32),
        jax.ShapeDtypeStruct((_B, _D, _D), jnp.float32),
    )
    out_map = lambda i: (jnp.maximum(i - 1, 0), 0, 0)
    return pl.pallas_call(
        _body,
        grid=grid,
        in_specs=[
            pl.BlockSpec(memory_space=pltpu.SMEM),
            pl.BlockSpec((_D, _D), lambda i: (0, 0)),
            pl.BlockSpec((_N, _D, _D), lambda i: (0, 0, 0)),
        ],
        out_specs=[
            pl.BlockSpec((_BB, _D, _D), out_map),
            pl.BlockSpec((_BB, _D, _D), out_map),
        ],
        out_shape=out_shape,
        scratch_shapes=[pltpu.VMEM((_N + 1, _D, _D), jnp.float32)],
    )(env_idx, W_adj, env_deltas)


def kernel(env_idx, W_adj, env_deltas):
    return _run(env_idx, W_adj, env_deltas)


# speedup vs baseline: 1.6254x; 1.1611x over previous
import jax
import jax.numpy as jnp
from jax.experimental import pallas as pl
from jax.experimental.pallas import tpu as pltpu

_D = 128
_N = 100
_B = 1024
_BB = 32
_TEMP = 1.0


def _body(env_idx_ref, w_ref, deltas_ref, a_ref, wb_ref, ta_ref):
    i = pl.program_id(0)

    @pl.when(i == 0)
    def _tables():
        w = w_ref[...]
        row = jax.lax.broadcasted_iota(jnp.int32, (_D, _D), 0)
        col = jax.lax.broadcasted_iota(jnp.int32, (_D, _D), 1)
        mask = jnp.where(row == col, 0.0, 1.0)
        ta_ref[0:_N] = jax.nn.sigmoid((w[None] + deltas_ref[...]) * (1.0 / _TEMP)) * mask[None]
        ta_ref[_N] = jax.nn.sigmoid(w * (1.0 / _TEMP)) * mask

    @pl.when(i > 0)
    def _emit():
        w = w_ref[...]
        base = (i - 1) * _BB
        for j in range(_BB):
            e = env_idx_ref[base + j]
            idx = jnp.clip(e, 0, _N - 1)
            valid = e < _N
            wb_ref[j] = w + jnp.where(valid, 1.0, 0.0) * deltas_ref[idx]
            a_ref[j] = ta_ref[jnp.where(valid, idx, _N)]


@jax.jit
def _run(env_idx, W_adj, env_deltas):
    grid = (1 + _B // _BB,)
    out_shape = (
        jax.ShapeDtypeStruct((_B, _D, _D), jnp.float32),
        jax.ShapeDtypeStruct((_B, _D, _D), jnp.float32),
    )
    out_map = lambda i: (jnp.maximum(i - 1, 0), 0, 0)
    return pl.pallas_call(
        _body,
        grid=grid,
        in_specs=[
            pl.BlockSpec(memory_space=pltpu.SMEM),
            pl.BlockSpec((_D, _D), lambda i: (0, 0)),
            pl.BlockSpec((_N, _D, _D), lambda i: (0, 0, 0)),
        ],
        out_specs=[
            pl.BlockSpec((_BB, _D, _D), out_map),
            pl.BlockSpec((_BB, _D, _D), out_map),
        ],
        out_shape=out_shape,
        scratch_shapes=[pltpu.VMEM((_N + 1, _D, _D), jnp.float32)],
    )(env_idx, W_adj, env_deltas)


def kernel(env_idx, W_adj, env_deltas):
    return _run(env_idx, W_adj, env_deltas)
